# Initial kernel scaffold; baseline (speedup 1.0000x reference)
import jax
import jax.numpy as jnp
from jax.experimental import pallas as pl


def kernel(mode, edge_index, entity_embed, W1, b1, W2, b2):
    raise NotImplementedError("write your pallas kernel here")


# speedup vs baseline: 7.1940x; 7.1940x over previous
import functools

import jax
import jax.numpy as jnp
from jax import lax
from jax.experimental import pallas as pl
from jax.experimental.pallas import tpu as pltpu
from jax.experimental.pallas import tpu_sc as plsc

N_NODES = 10000
N_EDGES = 320000
DIM = 128

NC = 2
NS = 16
EDGES_PER_TILE = N_EDGES // (NC * NS)
CHUNK = 80
SUP = 5
SCHUNK = 25
NCHUNK = SUP * SCHUNK
LGRP = CHUNK // 16
N_PAD = 10240
ROWS_PER_TILE = N_PAD // NS
ZROWS = 64
SHIFT = 14
MASK = (1 << SHIFT) - 1


def _sc_segment_sum(pk_r, entity_embed, zrows):
    mesh = plsc.VectorSubcoreMesh(core_axis_name="c", subcore_axis_name="s")

    @functools.partial(
        pl.kernel,
        mesh=mesh,
        out_type=jax.ShapeDtypeStruct((NC, N_PAD, DIM), jnp.float32),
        scratch_types=[
            pltpu.VMEM((SCHUNK, CHUNK), jnp.int32),
            pltpu.VMEM((SCHUNK, CHUNK), jnp.int32),
            pltpu.VMEM((SCHUNK, CHUNK), jnp.int32),
            pltpu.VMEM((CHUNK, DIM), jnp.float32),
            pltpu.VMEM((ZROWS, DIM), jnp.float32),
            pltpu.VMEM_SHARED((N_PAD, DIM), jnp.float32),
            pltpu.SemaphoreType.DMA,
        ],
    )
    def seg_sum(pk_hbm, ent_hbm, z_hbm, out_hbm,
                pk_v, src_v, dst_v, rows_v, zbuf_v, h_sh, sem):
        c = lax.axis_index("c")
        s = lax.axis_index("s")
        pltpu.sync_copy(z_hbm, zbuf_v)
        base = s * ROWS_PER_TILE
        for z in range(ROWS_PER_TILE // ZROWS):
            pltpu.sync_copy(zbuf_v, h_sh.at[pl.ds(base + z * ZROWS, ZROWS)])
        plsc.subcore_barrier()

        def sup_body(q, carry):
            pltpu.sync_copy(pk_hbm.at[c, s, q], pk_v)

            def unpack(i, c2):
                for j in range(LGRP):
                    p = pk_v[i, pl.ds(j * 16, 16)]
                    src_v[i, pl.ds(j * 16, 16)] = lax.shift_right_logical(p, SHIFT)
                    dst_v[i, pl.ds(j * 16, 16)] = lax.bitwise_and(p, MASK)
                return c2

            lax.fori_loop(0, SCHUNK, unpack, 0)

            def body(g, c2):
                pltpu.async_copy(ent_hbm.at[src_v.at[g]], rows_v, sem).wait()
                pltpu.sync_copy(rows_v, h_sh.at[dst_v.at[g]], add=True)
                return c2

            lax.fori_loop(0, SCHUNK, body, 0)
            return carry

        lax.fori_loop(0, SUP, sup_body, 0)
        plsc.subcore_barrier()
        for z in range(ROWS_PER_TILE // ZROWS):
            off = base + z * ZROWS
            pltpu.sync_copy(h_sh.at[pl.ds(off, ZROWS)], zbuf_v)
            pltpu.sync_copy(zbuf_v, out_hbm.at[c, pl.ds(off, ZROWS)])

    return seg_sum(pk_r, entity_embed, zrows)


def _tc_dense(hp, entity_embed, w1t, b1, w2t, b2):
    rows = 400
    grid = N_NODES // rows

    def body(hp0, hp1, e, w1, bb1, w2, bb2, o):
        h = hp0[...] + hp1[...]
        a = jnp.dot(h, w1[...], preferred_element_type=jnp.float32) + bb1[...]
        b = jnp.dot(e[...] * h, w2[...], preferred_element_type=jnp.float32) + bb2[...]
        o[...] = jnp.where(a >= 0, a, 0.01 * a) + jnp.where(b >= 0, b, 0.01 * b)

    blk = pl.BlockSpec((rows, DIM), lambda i: (i, 0))
    wblk = pl.BlockSpec((DIM, DIM), lambda i: (0, 0))
    bblk = pl.BlockSpec((1, DIM), lambda i: (0, 0))
    return pl.pallas_call(
        body,
        grid=(grid,),
        in_specs=[blk, blk, blk, wblk, bblk, wblk, bblk],
        out_specs=blk,
        out_shape=jax.ShapeDtypeStruct((N_NODES, DIM), jnp.float32),
    )(hp[0], hp[1], entity_embed, w1t, b1, w2t, b2)


def kernel(mode, edge_index, entity_embed, W1, b1, W2, b2):
    pk = jnp.bitwise_or(jnp.left_shift(edge_index[0], SHIFT), edge_index[1])
    pk_r = pk.reshape(NC, NS, SUP, SCHUNK, CHUNK)
    zrows = jnp.zeros((ZROWS, DIM), jnp.float32)
    hp = _sc_segment_sum(pk_r, entity_embed, zrows)
    return _tc_dense(hp, entity_embed,
                     W1.T, b1.reshape(1, DIM), W2.T, b2.reshape(1, DIM))


# speedup vs baseline: 8.7627x; 1.2181x over previous
import functools

import jax
import jax.numpy as jnp
from jax import lax
from jax.experimental import pallas as pl
from jax.experimental.pallas import tpu as pltpu
from jax.experimental.pallas import tpu_sc as plsc

N_NODES = 10000
N_EDGES = 320000
DIM = 128

NC = 2
NS = 16
EDGES_PER_TILE = N_EDGES // (NC * NS)
CHUNK = 80
SUP = 5
SCHUNK = 25
NCHUNK = SUP * SCHUNK
LGRP = CHUNK // 16
NB = 2
N_PAD = 10240
ROWS_PER_TILE = N_PAD // NS
ZROWS = 64
SHIFT = 14
MASK = (1 << SHIFT) - 1


def _sc_segment_sum(pk_r, entity_embed, zrows):
    mesh = plsc.VectorSubcoreMesh(core_axis_name="c", subcore_axis_name="s")

    @functools.partial(
        pl.kernel,
        mesh=mesh,
        out_type=jax.ShapeDtypeStruct((NC, N_PAD, DIM), jnp.float32),
        scratch_types=[
            pltpu.VMEM((SCHUNK, CHUNK), jnp.int32),
            pltpu.VMEM((SCHUNK, CHUNK), jnp.int32),
            pltpu.VMEM((SCHUNK, CHUNK), jnp.int32),
            pltpu.VMEM((NB, CHUNK, DIM), jnp.float32),
            pltpu.VMEM((ZROWS, DIM), jnp.float32),
            pltpu.VMEM_SHARED((N_PAD, DIM), jnp.float32),
        ] + [pltpu.SemaphoreType.DMA] * (2 * NB),
    )
    def seg_sum(pk_hbm, ent_hbm, z_hbm, out_hbm,
                pk_v, src_v, dst_v, rows_v, zbuf_v, h_sh, *sems):
        gsem = sems[:NB]
        ssem = sems[NB:]
        c = lax.axis_index("c")
        s = lax.axis_index("s")
        pltpu.sync_copy(z_hbm, zbuf_v)
        base = s * ROWS_PER_TILE
        for z in range(ROWS_PER_TILE // ZROWS):
            pltpu.sync_copy(zbuf_v, h_sh.at[pl.ds(base + z * ZROWS, ZROWS)])

        plsc.subcore_barrier()

        def gather(g, b):
            pltpu.async_copy(ent_hbm.at[src_v.at[g]], rows_v.at[b], gsem[b])

        def gather_wait(b):
            pltpu.make_async_copy(ent_hbm.at[src_v.at[0]], rows_v.at[b],
                                  gsem[b]).wait()

        def scatter(g, b):
            pltpu.async_copy(rows_v.at[b], h_sh.at[dst_v.at[g]], ssem[b],
                             add=True)

        def scatter_wait(b):
            pltpu.make_async_copy(rows_v.at[b], h_sh.at[dst_v.at[0]],
                                  ssem[b]).wait()

        def sup_body(q, carry):
            pltpu.sync_copy(pk_hbm.at[c, s, q], pk_v)

            def unpack(i, c2):
                for j in range(LGRP):
                    p = pk_v[i, pl.ds(j * 16, 16)]
                    src_v[i, pl.ds(j * 16, 16)] = lax.shift_right_logical(p, SHIFT)
                    dst_v[i, pl.ds(j * 16, 16)] = lax.bitwise_and(p, MASK)
                return c2

            lax.fori_loop(0, SCHUNK, unpack, 0)

            gather(0, 0)

            def step(t, c2):
                b = lax.rem(t, 2)

                @pl.when(b == 0)
                def _():
                    gather_wait(0)
                    scatter(t, 0)

                @pl.when(b == 1)
                def _():
                    gather_wait(1)
                    scatter(t, 1)

                @pl.when(t + 1 < SCHUNK)
                def _():
                    @pl.when(b == 0)
                    def _():
                        @pl.when(t > 0)
                        def _():
                            scatter_wait(1)
                        gather(t + 1, 1)

                    @pl.when(b == 1)
                    def _():
                        scatter_wait(0)
                        gather(t + 1, 0)

                return c2

            lax.fori_loop(0, SCHUNK, step, 0)
            scatter_wait(1)
            scatter_wait(0)
            return carry

        lax.fori_loop(0, SUP, sup_body, 0)
        plsc.subcore_barrier()
        for z in range(ROWS_PER_TILE // ZROWS):
            off = base + z * ZROWS
            pltpu.sync_copy(h_sh.at[pl.ds(off, ZROWS)], zbuf_v)
            pltpu.sync_copy(zbuf_v, out_hbm.at[c, pl.ds(off, ZROWS)])

    return seg_sum(pk_r, entity_embed, zrows)


def _tc_dense(hp, entity_embed, w1t, b1, w2t, b2):
    rows = 400
    grid = N_NODES // rows

    def body(hp0, hp1, e, w1, bb1, w2, bb2, o):
        h = hp0[...] + hp1[...]
        a = jnp.dot(h, w1[...], preferred_element_type=jnp.float32) + bb1[...]
        b = jnp.dot(e[...] * h, w2[...], preferred_element_type=jnp.float32) + bb2[...]
        o[...] = jnp.where(a >= 0, a, 0.01 * a) + jnp.where(b >= 0, b, 0.01 * b)

    blk = pl.BlockSpec((rows, DIM), lambda i: (i, 0))
    wblk = pl.BlockSpec((DIM, DIM), lambda i: (0, 0))
    bblk = pl.BlockSpec((1, DIM), lambda i: (0, 0))
    return pl.pallas_call(
        body,
        grid=(grid,),
        in_specs=[blk, blk, blk, wblk, bblk, wblk, bblk],
        out_specs=blk,
        out_shape=jax.ShapeDtypeStruct((N_NODES, DIM), jnp.float32),
    )(hp[0], hp[1], entity_embed, w1t, b1, w2t, b2)


def kernel(mode, edge_index, entity_embed, W1, b1, W2, b2):
    pk = jnp.bitwise_or(jnp.left_shift(edge_index[0], SHIFT), edge_index[1])
    pk_r = pk.reshape(NC, NS, SUP, SCHUNK, CHUNK)
    zrows = jnp.zeros((ZROWS, DIM), jnp.float32)
    hp = _sc_segment_sum(pk_r, entity_embed, zrows)
    return _tc_dense(hp, entity_embed,
                     W1.T, b1.reshape(1, DIM), W2.T, b2.reshape(1, DIM))


# speedup vs baseline: 10.5079x; 1.1992x over previous
import functools

import jax
import jax.numpy as jnp
from jax import lax
from jax.experimental import pallas as pl
from jax.experimental.pallas import tpu as pltpu
from jax.experimental.pallas import tpu_sc as plsc

N_NODES = 10000
N_EDGES = 320000
DIM = 128

NC = 2
NS = 16
EDGES_PER_TILE = N_EDGES // (NC * NS)
CHUNK = 80
SUP = 5
SCHUNK = 25
NCHUNK = SUP * SCHUNK
LGRP = CHUNK // 16
NB = 3
LA = NB - 1
N_PAD = 10240
ROWS_PER_TILE = N_PAD // NS
ZROWS = 64
SHIFT = 14
MASK = (1 << SHIFT) - 1


def _sc_segment_sum(pk_r, entity_embed, zrows):
    mesh = plsc.VectorSubcoreMesh(core_axis_name="c", subcore_axis_name="s")

    @functools.partial(
        pl.kernel,
        mesh=mesh,
        out_type=jax.ShapeDtypeStruct((NC, N_PAD, DIM), jnp.float32),
        scratch_types=[
            pltpu.VMEM((SCHUNK, CHUNK), jnp.int32),
            pltpu.VMEM((SCHUNK, CHUNK), jnp.int32),
            pltpu.VMEM((SCHUNK, CHUNK), jnp.int32),
            pltpu.VMEM((NB, CHUNK, DIM), jnp.float32),
            pltpu.VMEM_SHARED((N_PAD, DIM), jnp.float32),
        ] + [pltpu.SemaphoreType.DMA] * (2 * NB),
    )
    def seg_sum(pk_hbm, ent_hbm, z_hbm, out_hbm,
                pk_v, src_v, dst_v, rows_v, h_sh, *sems):
        gsem = sems[:NB]
        ssem = sems[NB:]
        c = lax.axis_index("c")
        s = lax.axis_index("s")
        base = s * ROWS_PER_TILE
        for z in range(ROWS_PER_TILE // ZROWS):
            pltpu.sync_copy(z_hbm, h_sh.at[pl.ds(base + z * ZROWS, ZROWS)])

        plsc.subcore_barrier()

        def gather(g, b):
            pltpu.async_copy(ent_hbm.at[src_v.at[g]], rows_v.at[b], gsem[b])

        def gather_wait(b):
            pltpu.make_async_copy(ent_hbm.at[src_v.at[0]], rows_v.at[b],
                                  gsem[b]).wait()

        def scatter(g, b):
            pltpu.async_copy(rows_v.at[b], h_sh.at[dst_v.at[g]], ssem[b],
                             add=True)

        def scatter_wait(b):
            pltpu.make_async_copy(rows_v.at[b], h_sh.at[dst_v.at[0]],
                                  ssem[b]).wait()

        def sup_body(q, carry):
            pltpu.sync_copy(pk_hbm.at[c, s, q], pk_v)

            def unpack(i, c2):
                for j in range(LGRP):
                    p = pk_v[i, pl.ds(j * 16, 16)]
                    src_v[i, pl.ds(j * 16, 16)] = lax.shift_right_logical(p, SHIFT)
                    dst_v[i, pl.ds(j * 16, 16)] = lax.bitwise_and(p, MASK)
                return c2

            lax.fori_loop(0, SCHUNK, unpack, 0)

            for g in range(LA):
                gather(g, g % NB)

            def step(t, c2):
                bt = lax.rem(t, NB)
                for k in range(NB):
                    @pl.when(bt == k)
                    def _(k=k):
                        gather_wait(k)
                        scatter(t, k)

                @pl.when(t + LA < SCHUNK)
                def _():
                    bp = lax.rem(t + LA, NB)
                    for k in range(NB):
                        @pl.when(bp == k)
                        def _(k=k):
                            @pl.when(t >= 1)
                            def _():
                                scatter_wait(k)
                            gather(t + LA, k)

                return c2

            lax.fori_loop(0, SCHUNK, step, 0)
            for k in range(NB):
                scatter_wait(k)
            return carry

        lax.fori_loop(0, SUP, sup_body, 0)
        plsc.subcore_barrier()
        pltpu.sync_copy(h_sh.at[pl.ds(base, ROWS_PER_TILE)],
                        out_hbm.at[c, pl.ds(base, ROWS_PER_TILE)])

    return seg_sum(pk_r, entity_embed, zrows)


def _tc_dense(hp, entity_embed, w1t, b1, w2t, b2):
    rows = 400
    grid = N_NODES // rows

    def body(hp0, hp1, e, w1, bb1, w2, bb2, o):
        h = hp0[...] + hp1[...]
        a = jnp.dot(h, w1[...], preferred_element_type=jnp.float32) + bb1[...]
        b = jnp.dot(e[...] * h, w2[...], preferred_element_type=jnp.float32) + bb2[...]
        o[...] = jnp.where(a >= 0, a, 0.01 * a) + jnp.where(b >= 0, b, 0.01 * b)

    blk = pl.BlockSpec((rows, DIM), lambda i: (i, 0))
    wblk = pl.BlockSpec((DIM, DIM), lambda i: (0, 0))
    bblk = pl.BlockSpec((1, DIM), lambda i: (0, 0))
    return pl.pallas_call(
        body,
        grid=(grid,),
        in_specs=[blk, blk, blk, wblk, bblk, wblk, bblk],
        out_specs=blk,
        out_shape=jax.ShapeDtypeStruct((N_NODES, DIM), jnp.float32),
    )(hp[0], hp[1], entity_embed, w1t, b1, w2t, b2)


def kernel(mode, edge_index, entity_embed, W1, b1, W2, b2):
    pk = jnp.bitwise_or(jnp.left_shift(edge_index[0], SHIFT), edge_index[1])
    pk_r = pk.reshape(NC, NS, SUP, SCHUNK, CHUNK)
    zrows = jnp.zeros((ZROWS, DIM), jnp.float32)
    hp = _sc_segment_sum(pk_r, entity_embed, zrows)
    return _tc_dense(hp, entity_embed,
                     W1.T, b1.reshape(1, DIM), W2.T, b2.reshape(1, DIM))
